# baseline (speedup 1.0000x reference)
import functools

import jax
import jax.numpy as jnp
from jax import lax
from jax.experimental import pallas as pl
from jax.experimental.pallas import tpu as pltpu
from jax.experimental.pallas import tpu_sc as plsc

N = 10000
E = 320000
D = 128
H = 256
L = 3

W = 80
NCHUNK = E // W
NSC = 16
EB = 4000
NB = 2000

_mesh = plsc.VectorSubcoreMesh(core_axis_name="core", subcore_axis_name="subcore",
                               num_cores=2, num_subcores=NSC)


@jax.jit
def _sc_gather(v, idx2):
  @pl.kernel(
      out_type=jax.ShapeDtypeStruct((2 * E, D // 2), jnp.int32),
      mesh=_mesh,
      compiler_params=pltpu.CompilerParams(use_tc_tiling_on_sc=False),
  )
  def k(v_hbm, i_hbm, o_hbm):
    def body(i_vmem, o_vmem):
      pltpu.sync_copy(v_hbm.at[i_vmem.at[0]], o_vmem)

    pltpu.emit_pipeline(
        body,
        grid=(2 * NCHUNK,),
        in_specs=[pl.BlockSpec((1, W), lambda i: (i, 0))],
        out_specs=[pl.BlockSpec((W, D // 2), lambda i: (i, 0))],
        core_axis_name=("core", "subcore"),
        dimension_semantics=(pltpu.PARALLEL,),
    )(i_hbm, o_hbm)

  return k(v, idx2)


def _bf16_to_i32(x):
  return lax.bitcast_convert_type(x.reshape(x.shape[0], -1, 2), jnp.int32)


def _i32_to_bf16(x):
  return lax.bitcast_convert_type(x, jnp.bfloat16).reshape(x.shape[0], -1)


@jax.jit
def _sc_scatter_add(de, dst2, zeros_nd):
  @pl.kernel(
      out_type=jax.ShapeDtypeStruct((2, N, D), jnp.float32),
      mesh=_mesh,
      scratch_types=[pltpu.VMEM_SHARED((N, D), jnp.float32)],
  )
  def k(de_hbm, i_hbm, z_hbm, o_hbm, acc):
    c = lax.axis_index("core")
    s = lax.axis_index("subcore")

    def _each_slice(fn):
      @pl.when(s < NSC - 1)
      def _():
        fn(pl.ds(s * 624, 624))

      @pl.when(s == NSC - 1)
      def _():
        fn(pl.ds((NSC - 1) * 624, N - (NSC - 1) * 624))

    _each_slice(lambda sl: pltpu.sync_copy(z_hbm.at[sl], acc.at[sl]))
    plsc.subcore_barrier()

    def body(de_vmem, i_vmem):
      pltpu.sync_copy(de_vmem, acc.at[i_vmem.at[0]], add=True)

    pltpu.emit_pipeline(
        body,
        grid=(NCHUNK,),
        in_specs=[pl.BlockSpec((W, D), lambda i: (i, 0)),
                  pl.BlockSpec((1, W), lambda i: (i, 0))],
        out_specs=[],
        core_axis_name=("core", "subcore"),
        dimension_semantics=(pltpu.PARALLEL,),
    )(de_hbm, i_hbm)

    plsc.subcore_barrier()
    _each_slice(lambda sl: pltpu.sync_copy(acc.at[sl], o_hbm.at[c].at[sl]))

  return k(de, dst2, zeros_nd)


def _edge_body(gs_ref, gd_ref, e_ref, w1_ref, b1_ref, w2_ref, b2_ref,
               de_ref, enew_ref):
  x = jnp.concatenate(
      [gs_ref[...], gd_ref[...], e_ref[...].astype(jnp.bfloat16)], axis=1)
  h = jnp.maximum(
      jnp.dot(x, w1_ref[...], preferred_element_type=jnp.float32)
      + b1_ref[...], 0.0)
  de = jnp.dot(h, w2_ref[...], preferred_element_type=jnp.float32) + b2_ref[...]
  de_ref[...] = de
  if enew_ref is not None:
    enew_ref[...] = e_ref[...] + de


@functools.partial(jax.jit, static_argnames=("want_enew",))
def _edge_mlp(g, e, w1, b1, w2, b2, want_enew=True):
  nblk = E // EB
  out_shape = [jax.ShapeDtypeStruct((E, D), jnp.float32)]
  if want_enew:
    out_shape.append(jax.ShapeDtypeStruct((E, D), jnp.float32))
  body = _edge_body if want_enew else (
      lambda gs, gd, e_, w1_, b1_, w2_, b2_, de_: _edge_body(
          gs, gd, e_, w1_, b1_, w2_, b2_, de_, None))
  outs = pl.pallas_call(
      body,
      grid=(nblk,),
      in_specs=[
          pl.BlockSpec((EB, D), lambda i: (i, 0)),
          pl.BlockSpec((EB, D), lambda i: (i + nblk, 0)),
          pl.BlockSpec((EB, D), lambda i: (i, 0)),
          pl.BlockSpec((3 * D, H), lambda i: (0, 0)),
          pl.BlockSpec((1, H), lambda i: (0, 0)),
          pl.BlockSpec((H, D), lambda i: (0, 0)),
          pl.BlockSpec((1, D), lambda i: (0, 0)),
      ],
      out_specs=[pl.BlockSpec((EB, D), lambda i: (i, 0))] * len(out_shape),
      out_shape=out_shape,
  )(g, g, e, w1, b1, w2, b2)
  return outs


def _node_body(v_ref, p_ref, w1_ref, b1_ref, w2_ref, b2_ref, dv_ref, vnew_ref,
               vnewbf_ref):
  agg = p_ref[0] + p_ref[1]
  x = jnp.concatenate([v_ref[...], agg], axis=1)
  h = jnp.maximum(
      jnp.dot(x, w1_ref[...], preferred_element_type=jnp.float32)
      + b1_ref[...], 0.0)
  dv = jnp.dot(h, w2_ref[...], preferred_element_type=jnp.float32) + b2_ref[...]
  dv_ref[...] = dv
  if vnew_ref is not None:
    vnew = v_ref[...] + dv
    vnew_ref[...] = vnew
    vnewbf_ref[...] = vnew.astype(jnp.bfloat16)


@functools.partial(jax.jit, static_argnames=("want_vnew",))
def _node_mlp(v, p, w1, b1, w2, b2, want_vnew=True):
  nblk = N // NB
  out_shape = [jax.ShapeDtypeStruct((N, D), jnp.float32)]
  if want_vnew:
    out_shape.append(jax.ShapeDtypeStruct((N, D), jnp.float32))
    out_shape.append(jax.ShapeDtypeStruct((N, D), jnp.bfloat16))
  body = _node_body if want_vnew else (
      lambda v_, p_, w1_, b1_, w2_, b2_, dv_: _node_body(
          v_, p_, w1_, b1_, w2_, b2_, dv_, None, None))
  outs = pl.pallas_call(
      body,
      grid=(nblk,),
      in_specs=[
          pl.BlockSpec((NB, D), lambda i: (i, 0)),
          pl.BlockSpec((2, NB, D), lambda i: (0, i, 0)),
          pl.BlockSpec((2 * D, H), lambda i: (0, 0)),
          pl.BlockSpec((1, H), lambda i: (0, 0)),
          pl.BlockSpec((H, D), lambda i: (0, 0)),
          pl.BlockSpec((1, D), lambda i: (0, 0)),
      ],
      out_specs=[pl.BlockSpec((NB, D), lambda i: (i, 0))] * len(out_shape),
      out_shape=out_shape,
  )(v, p, w1, b1, w2, b2)
  return outs


def kernel(node_embeddings, edge_embeddings, edge_index, batch,
           We1, be1, We2, be2, Wv1, bv1, Wv2, bv2):
  del batch
  src = edge_index[0]
  dst = edge_index[1]
  idx2 = jnp.concatenate([src, dst]).reshape(2 * NCHUNK, W)
  dst2 = dst.reshape(NCHUNK, W)
  zeros_nd = jnp.zeros((N, D), jnp.float32)

  v = node_embeddings
  vbf = node_embeddings.astype(jnp.bfloat16)
  e = edge_embeddings
  for i in range(L):
    last = i == L - 1
    g = _i32_to_bf16(_sc_gather(_bf16_to_i32(vbf), idx2))
    eouts = _edge_mlp(g, e, We1[i].astype(jnp.bfloat16), be1[i].reshape(1, H),
                      We2[i], be2[i].reshape(1, D), want_enew=not last)
    de = eouts[0]
    p = _sc_scatter_add(de, dst2, zeros_nd)
    nouts = _node_mlp(v, p, Wv1[i], bv1[i].reshape(1, H),
                      Wv2[i], bv2[i].reshape(1, D), want_vnew=not last)
    dv = nouts[0]
    if last:
      return (dv, de)
    e = eouts[1]
    v = nouts[1]
    vbf = nouts[2]


# speedup vs baseline: 3.1270x; 3.1270x over previous
import functools

import jax
import jax.numpy as jnp
from jax import lax
from jax.experimental import pallas as pl
from jax.experimental.pallas import tpu as pltpu
from jax.experimental.pallas import tpu_sc as plsc

N = 10000
E = 320000
D = 128
H = 256
L = 3

W = 80
NCHUNK = E // W
NSC = 16
EB = 4000
NB = 2000

_mesh = plsc.VectorSubcoreMesh(core_axis_name="core", subcore_axis_name="subcore",
                               num_cores=2, num_subcores=NSC)


@jax.jit
def _sc_gather(v, idx2):
  @pl.kernel(
      out_type=jax.ShapeDtypeStruct((2 * E, D), jnp.float32),
      mesh=_mesh,
  )
  def k(v_hbm, i_hbm, o_hbm):
    def body(i_vmem, o_vmem):
      pltpu.sync_copy(v_hbm.at[i_vmem.at[0]], o_vmem)

    pltpu.emit_pipeline(
        body,
        grid=(2 * NCHUNK,),
        in_specs=[pl.BlockSpec((1, W), lambda i: (i, 0))],
        out_specs=[pl.BlockSpec((W, D), lambda i: (i, 0))],
        core_axis_name=("core", "subcore"),
        dimension_semantics=(pltpu.PARALLEL,),
    )(i_hbm, o_hbm)

  return k(v, idx2)


@jax.jit
def _sc_scatter_add(de, dst2, zeros_nd):
  @pl.kernel(
      out_type=jax.ShapeDtypeStruct((2, N, D), jnp.float32),
      mesh=_mesh,
      scratch_types=[pltpu.VMEM_SHARED((N, D), jnp.float32)],
  )
  def k(de_hbm, i_hbm, z_hbm, o_hbm, acc):
    c = lax.axis_index("core")
    s = lax.axis_index("subcore")

    def _each_slice(fn):
      @pl.when(s < NSC - 1)
      def _():
        fn(pl.ds(s * 624, 624))

      @pl.when(s == NSC - 1)
      def _():
        fn(pl.ds((NSC - 1) * 624, N - (NSC - 1) * 624))

    _each_slice(lambda sl: pltpu.sync_copy(z_hbm.at[sl], acc.at[sl]))
    plsc.subcore_barrier()

    def body(de_vmem, i_vmem):
      pltpu.sync_copy(de_vmem, acc.at[i_vmem.at[0]], add=True)

    pltpu.emit_pipeline(
        body,
        grid=(NCHUNK,),
        in_specs=[pl.BlockSpec((W, D), lambda i: (i, 0)),
                  pl.BlockSpec((1, W), lambda i: (i, 0))],
        out_specs=[],
        core_axis_name=("core", "subcore"),
        dimension_semantics=(pltpu.PARALLEL,),
    )(de_hbm, i_hbm)

    plsc.subcore_barrier()
    _each_slice(lambda sl: pltpu.sync_copy(acc.at[sl], o_hbm.at[c].at[sl]))

  return k(de, dst2, zeros_nd)


def _edge_body(gs_ref, gd_ref, e_ref, w1_ref, b1_ref, w2_ref, b2_ref,
               de_ref, enew_ref):
  x = jnp.concatenate([gs_ref[...], gd_ref[...], e_ref[...]],
                      axis=1).astype(jnp.bfloat16)
  h = jnp.maximum(
      jnp.dot(x, w1_ref[...], preferred_element_type=jnp.float32)
      + b1_ref[...], 0.0)
  de = jnp.dot(h, w2_ref[...], preferred_element_type=jnp.float32) + b2_ref[...]
  de_ref[...] = de
  if enew_ref is not None:
    enew_ref[...] = (e_ref[...].astype(jnp.float32) + de).astype(jnp.bfloat16)


@functools.partial(jax.jit, static_argnames=("want_enew",))
def _edge_mlp(g, e, w1, b1, w2, b2, want_enew=True):
  nblk = E // EB
  out_shape = [jax.ShapeDtypeStruct((E, D), jnp.float32)]
  if want_enew:
    out_shape.append(jax.ShapeDtypeStruct((E, D), jnp.bfloat16))
  body = _edge_body if want_enew else (
      lambda gs, gd, e_, w1_, b1_, w2_, b2_, de_: _edge_body(
          gs, gd, e_, w1_, b1_, w2_, b2_, de_, None))
  outs = pl.pallas_call(
      body,
      grid=(nblk,),
      in_specs=[
          pl.BlockSpec((EB, D), lambda i: (i, 0)),
          pl.BlockSpec((EB, D), lambda i: (i + nblk, 0)),
          pl.BlockSpec((EB, D), lambda i: (i, 0)),
          pl.BlockSpec((3 * D, H), lambda i: (0, 0)),
          pl.BlockSpec((1, H), lambda i: (0, 0)),
          pl.BlockSpec((H, D), lambda i: (0, 0)),
          pl.BlockSpec((1, D), lambda i: (0, 0)),
      ],
      out_specs=[pl.BlockSpec((EB, D), lambda i: (i, 0))] * len(out_shape),
      out_shape=out_shape,
  )(g, g, e, w1, b1, w2, b2)
  return outs


def _node_body(v_ref, p_ref, w1_ref, b1_ref, w2_ref, b2_ref, dv_ref, vnew_ref):
  agg = p_ref[0] + p_ref[1]
  x = jnp.concatenate([v_ref[...], agg], axis=1)
  h = jnp.maximum(
      jnp.dot(x, w1_ref[...], preferred_element_type=jnp.float32)
      + b1_ref[...], 0.0)
  dv = jnp.dot(h, w2_ref[...], preferred_element_type=jnp.float32) + b2_ref[...]
  dv_ref[...] = dv
  if vnew_ref is not None:
    vnew_ref[...] = v_ref[...] + dv


@functools.partial(jax.jit, static_argnames=("want_vnew",))
def _node_mlp(v, p, w1, b1, w2, b2, want_vnew=True):
  nblk = N // NB
  out_shape = [jax.ShapeDtypeStruct((N, D), jnp.float32)]
  if want_vnew:
    out_shape.append(jax.ShapeDtypeStruct((N, D), jnp.float32))
  body = _node_body if want_vnew else (
      lambda v_, p_, w1_, b1_, w2_, b2_, dv_: _node_body(
          v_, p_, w1_, b1_, w2_, b2_, dv_, None))
  outs = pl.pallas_call(
      body,
      grid=(nblk,),
      in_specs=[
          pl.BlockSpec((NB, D), lambda i: (i, 0)),
          pl.BlockSpec((2, NB, D), lambda i: (0, i, 0)),
          pl.BlockSpec((2 * D, H), lambda i: (0, 0)),
          pl.BlockSpec((1, H), lambda i: (0, 0)),
          pl.BlockSpec((H, D), lambda i: (0, 0)),
          pl.BlockSpec((1, D), lambda i: (0, 0)),
      ],
      out_specs=[pl.BlockSpec((NB, D), lambda i: (i, 0))] * len(out_shape),
      out_shape=out_shape,
  )(v, p, w1, b1, w2, b2)
  return outs


def kernel(node_embeddings, edge_embeddings, edge_index, batch,
           We1, be1, We2, be2, Wv1, bv1, Wv2, bv2):
  del batch
  src = edge_index[0]
  dst = edge_index[1]
  idx2 = jnp.concatenate([src, dst]).reshape(2 * NCHUNK, W)
  dst2 = dst.reshape(NCHUNK, W)
  zeros_nd = jnp.zeros((N, D), jnp.float32)

  v = node_embeddings
  e = edge_embeddings
  for i in range(L):
    last = i == L - 1
    g = _sc_gather(v, idx2)
    eouts = _edge_mlp(g, e, We1[i].astype(jnp.bfloat16), be1[i].reshape(1, H),
                      We2[i], be2[i].reshape(1, D), want_enew=not last)
    de = eouts[0]
    p = _sc_scatter_add(de, dst2, zeros_nd)
    nouts = _node_mlp(v, p, Wv1[i], bv1[i].reshape(1, H),
                      Wv2[i], bv2[i].reshape(1, D), want_vnew=not last)
    dv = nouts[0]
    if last:
      return (dv, de)
    e = eouts[1]
    v = nouts[1]
